# baseline (speedup 1.0000x reference)
import functools

import jax
import jax.numpy as jnp
from jax import lax
from jax.experimental import pallas as pl
from jax.experimental.pallas import tpu as pltpu
from jax.experimental.pallas import tpu_sc as plsc

_B = 4096
_S = 200
_D = 64
_NC = 2
_NS = 16
_NW = _NC * _NS
_NB = _B // _NW
_G = 40
_NG = _S // _G


def _gather_body(idx_hbm, table_hbm, out_hbm, idx_v, rows_v, sem):
    wid = lax.axis_index("s") * _NC + lax.axis_index("c")
    b0 = wid * _NB
    pltpu.sync_copy(idx_hbm.at[pl.ds(b0, _NB)], idx_v)

    def step(c, carry):
        copies = [
            pltpu.async_copy(
                table_hbm.at[idx_v.at[c, pl.ds(j * _G, _G)]],
                rows_v.at[pl.ds(j * _G, _G)],
                sem,
            )
            for j in range(_NG)
        ]
        for cp in copies:
            cp.wait()
        pltpu.sync_copy(rows_v, out_hbm.at[b0 + c])
        return carry

    lax.fori_loop(0, _NB, step, 0)


_mesh = plsc.VectorSubcoreMesh(core_axis_name="c", subcore_axis_name="s")

_gather = functools.partial(
    pl.kernel,
    out_type=jax.ShapeDtypeStruct((_B, _S, _D), jnp.float32),
    mesh=_mesh,
    scratch_types=[
        pltpu.VMEM((_NB, _S), jnp.int32),
        pltpu.VMEM((_S, _D), jnp.float32),
        pltpu.SemaphoreType.DMA,
    ],
    compiler_params=pltpu.CompilerParams(use_tc_tiling_on_sc=False),
)(_gather_body)


def kernel(gene_indices, table):
    return _gather(gene_indices, table)
